# baseline (speedup 1.0000x reference)
import jax
import jax.numpy as jnp
from jax.experimental import pallas as pl
from jax.experimental.pallas import tpu as pltpu

_BS = 1024


def _body(x_hbm, pos_hbm, out_hbm, buf, posbuf, in_sems, out_sems, pos_sems):
    B = 4
    s = pl.program_id(0)
    b = pl.program_id(1)
    k = s * B + b
    nsteps = pl.num_programs(0) * B

    def in_copy(kk):
        ss, bb = kk // B, kk % B
        return pltpu.make_async_copy(
            x_hbm.at[bb, pl.ds(ss * _BS, _BS), :], buf.at[bb], in_sems.at[bb]
        )

    def out_copy(kk):
        ss, bb = kk // B, kk % B
        return pltpu.make_async_copy(
            buf.at[bb], out_hbm.at[bb, pl.ds(ss * _BS, _BS), :], out_sems.at[bb]
        )

    def pos_copy(ss):
        return pltpu.make_async_copy(
            pos_hbm.at[pl.ds(ss * _BS, _BS), :],
            posbuf.at[ss % 2],
            pos_sems.at[ss % 2],
        )

    @pl.when(k == 0)
    def _():
        pos_copy(0).start()
        in_copy(0).start()
        in_copy(1).start()

    @pl.when(k <= nsteps - 3)
    def _():
        @pl.when(k >= 2)
        def _():
            out_copy(k - 2).wait()

        in_copy(k + 2).start()

    @pl.when(jnp.logical_and(b == 2, s + 1 < pl.num_programs(0)))
    def _():
        pos_copy(s + 1).start()

    @pl.when(b == 0)
    def _():
        pos_copy(s).wait()

    in_copy(k).wait()
    buf[b] = buf[b] + posbuf[s % 2]
    out_copy(k).start()

    @pl.when(k == nsteps - 1)
    def _():
        for d in range(4):
            out_copy(nsteps - 4 + d).wait()


def kernel(x, pos_embedding):
    B, S, D = x.shape
    grid = (S // _BS, B)
    return pl.pallas_call(
        _body,
        grid=grid,
        in_specs=[
            pl.BlockSpec(memory_space=pltpu.MemorySpace.HBM),
            pl.BlockSpec(memory_space=pltpu.MemorySpace.HBM),
        ],
        out_specs=pl.BlockSpec(memory_space=pltpu.MemorySpace.HBM),
        out_shape=jax.ShapeDtypeStruct((B, S, D), x.dtype),
        scratch_shapes=[
            pltpu.VMEM((B, _BS, D), jnp.float32),
            pltpu.VMEM((2, _BS, D), jnp.float32),
            pltpu.SemaphoreType.DMA((B,)),
            pltpu.SemaphoreType.DMA((B,)),
            pltpu.SemaphoreType.DMA((2,)),
        ],
        compiler_params=pltpu.CompilerParams(
            dimension_semantics=("arbitrary", "arbitrary"),
        ),
    )(x, pos_embedding[:S])


# speedup vs baseline: 1.0026x; 1.0026x over previous
import jax
import jax.numpy as jnp
from jax.experimental import pallas as pl
from jax.experimental.pallas import tpu as pltpu

_BS = 2048


def _body(x_hbm, pos_hbm, out_hbm, buf, posbuf, in_sems, out_sems, pos_sems):
    B = 4
    s = pl.program_id(0)
    b = pl.program_id(1)
    k = s * B + b
    nsteps = pl.num_programs(0) * B

    def in_copy(kk):
        ss, bb = kk // B, kk % B
        return pltpu.make_async_copy(
            x_hbm.at[bb, pl.ds(ss * _BS, _BS), :], buf.at[bb], in_sems.at[bb]
        )

    def out_copy(kk):
        ss, bb = kk // B, kk % B
        return pltpu.make_async_copy(
            buf.at[bb], out_hbm.at[bb, pl.ds(ss * _BS, _BS), :], out_sems.at[bb]
        )

    def pos_copy(ss):
        return pltpu.make_async_copy(
            pos_hbm.at[pl.ds(ss * _BS, _BS), :],
            posbuf.at[ss % 2],
            pos_sems.at[ss % 2],
        )

    @pl.when(k == 0)
    def _():
        pos_copy(0).start()
        in_copy(0).start()
        in_copy(1).start()

    @pl.when(k <= nsteps - 3)
    def _():
        @pl.when(k >= 2)
        def _():
            out_copy(k - 2).wait()

        in_copy(k + 2).start()

    @pl.when(jnp.logical_and(b == 2, s + 1 < pl.num_programs(0)))
    def _():
        pos_copy(s + 1).start()

    @pl.when(b == 0)
    def _():
        pos_copy(s).wait()

    in_copy(k).wait()
    buf[b] = buf[b] + posbuf[s % 2]
    out_copy(k).start()

    @pl.when(k == nsteps - 1)
    def _():
        for d in range(4):
            out_copy(nsteps - 4 + d).wait()


def kernel(x, pos_embedding):
    B, S, D = x.shape
    grid = (S // _BS, B)
    return pl.pallas_call(
        _body,
        grid=grid,
        in_specs=[
            pl.BlockSpec(memory_space=pltpu.MemorySpace.HBM),
            pl.BlockSpec(memory_space=pltpu.MemorySpace.HBM),
        ],
        out_specs=pl.BlockSpec(memory_space=pltpu.MemorySpace.HBM),
        out_shape=jax.ShapeDtypeStruct((B, S, D), x.dtype),
        scratch_shapes=[
            pltpu.VMEM((B, _BS, D), jnp.float32),
            pltpu.VMEM((2, _BS, D), jnp.float32),
            pltpu.SemaphoreType.DMA((B,)),
            pltpu.SemaphoreType.DMA((B,)),
            pltpu.SemaphoreType.DMA((2,)),
        ],
        compiler_params=pltpu.CompilerParams(
            dimension_semantics=("arbitrary", "arbitrary"),
        ),
    )(x, pos_embedding[:S])
